# Initial kernel scaffold; baseline (speedup 1.0000x reference)
import jax
import jax.numpy as jnp
from jax.experimental import pallas as pl


def kernel(x, linear_w, bias, ffm_w):
    raise NotImplementedError("write your pallas kernel here")


# speedup vs baseline: 3.6397x; 3.6397x over previous
import functools

import numpy as np
import jax
import jax.numpy as jnp
from jax import lax
from jax.experimental import pallas as pl
from jax.experimental.pallas import tpu as pltpu
from jax.experimental.pallas import tpu_sc as plsc

_F = 26
_PER_FIELD = 3847
_V = _F * _PER_FIELD
_D = 16
_B = 4096
_OFF = np.array((0, *np.cumsum([_PER_FIELD] * _F)[:-1]), dtype=np.int32)

_PAIRS = [(i, j) for i in range(_F - 1) for j in range(i + 1, _F)]
_NPAIR = len(_PAIRS)
_NENT = 2 * _NPAIR
_NENT_PAD = 656

_CMAP = np.zeros(_NENT_PAD, np.int32)
_FMAP = np.zeros(_NENT_PAD, np.int32)
for _p, (_i, _j) in enumerate(_PAIRS):
    _CMAP[_p] = _j * _V + _OFF[_i]
    _FMAP[_p] = _i
    _CMAP[_NPAIR + _p] = _i * _V + _OFF[_j]
    _FMAP[_NPAIR + _p] = _j

_LCMAP = np.zeros(32, np.int32)
_LFMAP = np.zeros(32, np.int32)
_LCMAP[:_F] = _OFF
_LFMAP[:_F] = np.arange(_F)

_NC = 2
_NS = 16
_NW = _NC * _NS
_S = _B // _NW
_NBUF = 2


def _body(x_hbm, cmap_hbm, fmap_hbm, lcmap_hbm, lfmap_hbm, fw_hbm, lw_hbm,
          out_hbm, x_v, cmap_v, fmap_v, lcmap_v, lfmap_v, idx_v, lidx_v,
          rows_v, lrows_v, out_v, sems, lsems):
    wid = lax.axis_index("s") * _NC + lax.axis_index("c")
    base = wid * _S
    pltpu.sync_copy(x_hbm.at[pl.ds(base * _F, _S * _F)], x_v)
    pltpu.sync_copy(cmap_hbm, cmap_v)
    pltpu.sync_copy(fmap_hbm, fmap_v)
    pltpu.sync_copy(lcmap_hbm, lcmap_v)
    pltpu.sync_copy(lfmap_hbm, lfmap_v)

    lane = lax.broadcasted_iota(jnp.int32, (16,), 0)

    def fire(s, b):
        srow = jnp.full((16,), s * _F, jnp.int32)
        for c in range(_NENT_PAD // 16):
            sl = pl.ds(c * 16, 16)
            xv = plsc.load_gather(x_v, [srow + fmap_v[sl]])
            idx_v[b, sl] = cmap_v[sl] + xv
        for c in range(2):
            sl = pl.ds(c * 16, 16)
            lidx_v[b, sl] = lcmap_v[sl] + plsc.load_gather(
                x_v, [srow + lfmap_v[sl]])
        cps = []
        for c in range(5):
            cps.append(pltpu.async_copy(
                fw_hbm.at[idx_v.at[b, pl.ds(c * 128, 128)]],
                rows_v.at[b, pl.ds(c * 128, 128)], sems.at[b]))
        cps.append(pltpu.async_copy(
            fw_hbm.at[idx_v.at[b, pl.ds(640, 16)]],
            rows_v.at[b, pl.ds(640, 16)], sems.at[b]))
        cps.append(pltpu.async_copy(lw_hbm.at[lidx_v.at[b]],
                                    lrows_v.at[b], lsems.at[b]))
        return cps

    def drain(s, b):
        for c in range(5):
            pltpu.make_async_copy(
                fw_hbm.at[idx_v.at[b, pl.ds(c * 128, 128)]],
                rows_v.at[b, pl.ds(c * 128, 128)], sems.at[b]).wait()
        pltpu.make_async_copy(
            fw_hbm.at[idx_v.at[b, pl.ds(640, 16)]],
            rows_v.at[b, pl.ds(640, 16)], sems.at[b]).wait()
        pltpu.make_async_copy(lw_hbm.at[lidx_v.at[b]], lrows_v.at[b],
                              lsems.at[b]).wait()

        def red(p, acc):
            return acc + rows_v[b, p] * rows_v[b, _NPAIR + p]

        acc = lax.fori_loop(0, _NPAIR, red, jnp.zeros((16,), jnp.float32),
                            unroll=4)
        lin = lrows_v[b, pl.ds(0, 16)] + jnp.where(
            lane < _F - 16, lrows_v[b, pl.ds(16, 16)], 0.0)
        val = jnp.sum(acc) + jnp.sum(lin)
        plsc.store_scatter(out_v, [jnp.full((16,), s, jnp.int32)],
                           jnp.full((16,), val, jnp.float32),
                           mask=lane == 0)

    for b in range(_NBUF):
        fire(b, b)

    def step(g, _):
        s = g * _NBUF
        for b in range(_NBUF):
            nxt = s + b + _NBUF
            drain(s + b, b)
            fire(jnp.minimum(nxt, _S - 1), b)
        return 0

    lax.fori_loop(0, _S // _NBUF - 1, step, 0)
    s_last = _S - _NBUF
    for b in range(_NBUF):
        drain(s_last + b, b)

    pltpu.sync_copy(out_v, out_hbm.at[pl.ds(base, _S)])


@jax.jit
def _ffm_call(x, fw, lw):
    mesh = plsc.VectorSubcoreMesh(core_axis_name="c", subcore_axis_name="s",
                                  num_cores=_NC, num_subcores=_NS)
    call = pl.kernel(
        _body,
        out_type=jax.ShapeDtypeStruct((_B,), jnp.float32),
        mesh=mesh,
        compiler_params=pltpu.CompilerParams(needs_layout_passes=False,
                                             use_tc_tiling_on_sc=False),
        scratch_types=[
            pltpu.VMEM((_S * _F,), jnp.int32),
            pltpu.VMEM((_NENT_PAD,), jnp.int32),
            pltpu.VMEM((_NENT_PAD,), jnp.int32),
            pltpu.VMEM((32,), jnp.int32),
            pltpu.VMEM((32,), jnp.int32),
            pltpu.VMEM((_NBUF, _NENT_PAD), jnp.int32),
            pltpu.VMEM((_NBUF, 32), jnp.int32),
            pltpu.VMEM((_NBUF, _NENT_PAD, _D), jnp.float32),
            pltpu.VMEM((_NBUF, 32), jnp.float32),
            pltpu.VMEM((_S,), jnp.float32),
            pltpu.SemaphoreType.DMA((_NBUF,)),
            pltpu.SemaphoreType.DMA((_NBUF,)),
        ],
    )
    return call(x, jnp.asarray(_CMAP), jnp.asarray(_FMAP),
                jnp.asarray(_LCMAP), jnp.asarray(_LFMAP), fw, lw)


def kernel(x, linear_w, bias, ffm_w):
    x = x.astype(jnp.int32).reshape(_B * _F)
    fw = ffm_w.reshape(_F * _V, _D)
    lw = linear_w.reshape(_V)
    out = _ffm_call(x, fw, lw)
    return out + bias[0]
